# baseline (speedup 1.0000x reference)
import jax
import jax.numpy as jnp
from jax import lax
from jax.experimental import pallas as pl
from jax.experimental.pallas import tpu as pltpu
from jax.experimental.pallas import tpu_sc as plsc

_ENC_HID = 64
_DEC_HID = 64
_RNN_LAYERS = 2
_BATCH = 16384
_PAD = 128

_NC = 2
_NS = 16
_NW = _NC * _NS
_BPW = _BATCH // _NW
_WIN = 256
_VG = 16


def _fire_window(tbl, idx, lo, buf, sem):
    copies = []
    for g in range(_WIN // _VG):
        iv = idx[pl.ds(lo + g * _VG, _VG)]
        copies.append(pltpu.async_copy(
            tbl.at[iv], buf.at[pl.ds(g * _VG, _VG)], sem))
    return copies


def _sc_gather_body(idx_hbm, tbl,
                    out,
                    idx_v, buf_a, buf_b, gsem_a, gsem_b, ssem):
    wid = lax.axis_index("s") * _NC + lax.axis_index("c")
    base = wid * _BPW

    pltpu.sync_copy(idx_hbm.at[pl.ds(base, _BPW)], idx_v)

    ca = _fire_window(tbl, idx_v, 0, buf_a, gsem_a)
    cb = _fire_window(tbl, idx_v, _WIN, buf_b, gsem_b)
    for c in ca:
        c.wait()
    sa = pltpu.async_copy(buf_a, out.at[pl.ds(base, _WIN)], ssem)
    for c in cb:
        c.wait()
    sb = pltpu.async_copy(buf_b, out.at[pl.ds(base + _WIN, _WIN)], ssem)
    sa.wait()
    sb.wait()


@jax.jit
def _sc_gather1(idx, t128):
    padded = jax.ShapeDtypeStruct((_BATCH, _PAD), jnp.float32)
    mesh = plsc.VectorSubcoreMesh(core_axis_name="c", subcore_axis_name="s")
    return pl.kernel(
        _sc_gather_body,
        mesh=mesh,
        compiler_params=pltpu.CompilerParams(use_tc_tiling_on_sc=True),
        out_type=padded,
        scratch_types=[
            pltpu.VMEM((_BPW,), jnp.int32),
            pltpu.VMEM((_WIN, _PAD), jnp.float32),
            pltpu.VMEM((_WIN, _PAD), jnp.float32),
            pltpu.SemaphoreType.DMA,
            pltpu.SemaphoreType.DMA,
            pltpu.SemaphoreType.DMA,
        ],
    )(idx, t128)


_PREP_LANES = 6400
_PREP_GRID = -(-100000 // _PREP_LANES)


def _prep_body(ut_ref, u2_ref):
    u2_ref[:, :_ENC_HID] = ut_ref[...].T


@jax.jit
def _tc_prep1(tT):
    t2 = jax.ShapeDtypeStruct((100000, _PAD), jnp.float32)
    return pl.pallas_call(
        _prep_body,
        grid=(_PREP_GRID,),
        in_specs=[pl.BlockSpec((_ENC_HID, _PREP_LANES), lambda i: (0, i))],
        out_specs=pl.BlockSpec((_PREP_LANES, _PAD), lambda i: (i, 0)),
        out_shape=t2,
    )(tT)


def _dense_body(u_ref, i_ref, code_ref, rtab_ref,
                w_ref, b_ref, h_ref, ue_ref, ie_ref, re_ref):
    up = u_ref[...]
    ip = i_ref[...]
    rat = code_ref[...]
    u64 = up[:, :_ENC_HID]
    i64 = ip[:, :_ENC_HID]
    r_e = jnp.zeros(u64.shape, jnp.float32)
    for k in range(6):
        r_e = r_e + jnp.where(rat == k, 1.0, 0.0) * rtab_ref[k, :][None, :]
    cat = jnp.concatenate([u64, i64, r_e], axis=1)
    acc = jnp.dot(cat, w_ref[...], preferred_element_type=jnp.float32)
    h_ref[...] = jnp.tanh(acc + b_ref[...]).T
    ue_ref[...] = u64.T
    ie_ref[...] = i64.T
    re_ref[...] = r_e.T


@jax.jit
def _tc_dense(u_p, i_p, code, rtab, W, b2d):
    blk = 2048
    grid = (_BATCH // blk,)
    pad_spec = pl.BlockSpec((blk, _PAD), lambda i: (i, 0))
    emb_t_shape = jax.ShapeDtypeStruct((_ENC_HID, _BATCH), jnp.float32)
    return pl.pallas_call(
        _dense_body,
        grid=grid,
        in_specs=[
            pad_spec, pad_spec,
            pl.BlockSpec((blk, 1), lambda i: (i, 0)),
            pl.BlockSpec((6, _ENC_HID), lambda i: (0, 0)),
            pl.BlockSpec((3 * _ENC_HID, _DEC_HID * _RNN_LAYERS), lambda i: (0, 0)),
            pl.BlockSpec((1, _DEC_HID * _RNN_LAYERS), lambda i: (0, 0)),
        ],
        out_specs=[
            pl.BlockSpec((_DEC_HID * _RNN_LAYERS, blk), lambda i: (0, i)),
            pl.BlockSpec((_ENC_HID, blk), lambda i: (0, i)),
            pl.BlockSpec((_ENC_HID, blk), lambda i: (0, i)),
            pl.BlockSpec((_ENC_HID, blk), lambda i: (0, i)),
        ],
        out_shape=[
            jax.ShapeDtypeStruct((_DEC_HID * _RNN_LAYERS, _BATCH), jnp.float32),
            emb_t_shape, emb_t_shape, emb_t_shape,
        ],
    )(u_p, i_p, code, rtab, W, b2d)


def kernel(user, item, rating, user_table, item_table, rating_table, W, b):
    user = user.astype(jnp.int32)
    item = item.astype(jnp.int32)
    rating = rating.astype(jnp.int32)
    u128 = _tc_prep1(user_table.T)
    u_p = _sc_gather1(user, u128)
    i128 = _tc_prep1(item_table.T)
    i_p = _sc_gather1(item, i128)
    h_t, ue_t, ie_t, re_t = _tc_dense(u_p, i_p, rating.reshape(-1, 1),
                                      rating_table, W, b.reshape(1, -1))
    hidden = h_t.T.reshape(-1, _DEC_HID, _RNN_LAYERS)
    return (hidden, ue_t.T, ie_t.T, re_t.T)


# speedup vs baseline: 1.1267x; 1.1267x over previous
import jax
import jax.numpy as jnp
from jax import lax
from jax.experimental import pallas as pl
from jax.experimental.pallas import tpu as pltpu
from jax.experimental.pallas import tpu_sc as plsc

_ENC_HID = 64
_DEC_HID = 64
_RNN_LAYERS = 2
_BATCH = 16384
_PAD = 128
_NRAT = 6

_NC = 2
_NS = 16
_NW = _NC * _NS
_BPW = _BATCH // _NW
_WIN = 256
_VG = 16


def _fire_window(tbl, idx, lo, buf, sem):
    copies = []
    for g in range(_WIN // _VG):
        iv = idx[pl.ds(lo + g * _VG, _VG)]
        copies.append(pltpu.async_copy(
            tbl.at[iv], buf.at[pl.ds(g * _VG, _VG)], sem))
    return copies


def _sc_gather_body(user_hbm, item_hbm, user_tbl, item_tbl,
                    u_out, i_out,
                    u_idx, i_idx, buf_a, buf_b, gsem_a, gsem_b, ssem):
    wid = lax.axis_index("s") * _NC + lax.axis_index("c")
    base = wid * _BPW

    pltpu.sync_copy(user_hbm.at[pl.ds(base, _BPW)], u_idx)
    pltpu.sync_copy(item_hbm.at[pl.ds(base, _BPW)], i_idx)

    ca = _fire_window(user_tbl, u_idx, 0, buf_a, gsem_a)
    cb = _fire_window(user_tbl, u_idx, _WIN, buf_b, gsem_b)
    for c in ca:
        c.wait()
    sa = pltpu.async_copy(buf_a, u_out.at[pl.ds(base, _WIN)], ssem)
    for c in cb:
        c.wait()
    sb = pltpu.async_copy(buf_b, u_out.at[pl.ds(base + _WIN, _WIN)], ssem)

    sa.wait()
    ca = _fire_window(item_tbl, i_idx, 0, buf_a, gsem_a)
    sb.wait()
    cb = _fire_window(item_tbl, i_idx, _WIN, buf_b, gsem_b)
    for c in ca:
        c.wait()
    sa = pltpu.async_copy(buf_a, i_out.at[pl.ds(base, _WIN)], ssem)
    for c in cb:
        c.wait()
    sb = pltpu.async_copy(buf_b, i_out.at[pl.ds(base + _WIN, _WIN)], ssem)
    sa.wait()
    sb.wait()


@jax.jit
def _sc_gather(user, item, user_t128, item_t128):
    padded = jax.ShapeDtypeStruct((_BATCH, _PAD), jnp.float32)
    mesh = plsc.VectorSubcoreMesh(core_axis_name="c", subcore_axis_name="s")
    return pl.kernel(
        _sc_gather_body,
        mesh=mesh,
        compiler_params=pltpu.CompilerParams(use_tc_tiling_on_sc=True),
        out_type=(padded, padded),
        scratch_types=[
            pltpu.VMEM((_BPW,), jnp.int32),
            pltpu.VMEM((_BPW,), jnp.int32),
            pltpu.VMEM((_WIN, _PAD), jnp.float32),
            pltpu.VMEM((_WIN, _PAD), jnp.float32),
            pltpu.SemaphoreType.DMA,
            pltpu.SemaphoreType.DMA,
            pltpu.SemaphoreType.DMA,
        ],
    )(user, item, user_t128, item_t128)


_PREP_LANES = 6400
_PREP_GRID = -(-100000 // _PREP_LANES)


def _prep_body(ut_ref, it_ref, u2_ref, i2_ref):
    u2_ref[:, :_ENC_HID] = ut_ref[...].T
    i2_ref[:, :_ENC_HID] = it_ref[...].T


@jax.jit
def _tc_prep(user_tT, item_tT):
    t2 = jax.ShapeDtypeStruct((100000, _PAD), jnp.float32)
    tin_spec = pl.BlockSpec((_ENC_HID, _PREP_LANES), lambda i: (0, i))
    tout_spec = pl.BlockSpec((_PREP_LANES, _PAD), lambda i: (i, 0))
    return pl.pallas_call(
        _prep_body,
        grid=(_PREP_GRID,),
        in_specs=[tin_spec, tin_spec],
        out_specs=[tout_spec, tout_spec],
        out_shape=[t2, t2],
    )(user_tT, item_tT)


def _dense_body(u_ref, i_ref, oh_ref, rtab_ref, wt_ref, b_ref,
                h_ref, ue_ref, ie_ref, re_ref):
    ue_t = u_ref[...][:, :_ENC_HID].T
    ie_t = i_ref[...][:, :_ENC_HID].T
    re_t = jnp.dot(rtab_ref[...].T, oh_ref[...],
                   preferred_element_type=jnp.float32)
    wt = wt_ref[...]
    acc = (jnp.dot(wt[:, :_ENC_HID], ue_t, preferred_element_type=jnp.float32)
           + jnp.dot(wt[:, _ENC_HID:2 * _ENC_HID], ie_t,
                     preferred_element_type=jnp.float32)
           + jnp.dot(wt[:, 2 * _ENC_HID:], re_t,
                     preferred_element_type=jnp.float32))
    h_ref[...] = jnp.tanh(acc + b_ref[...])
    ue_ref[...] = ue_t
    ie_ref[...] = ie_t
    re_ref[...] = re_t


@jax.jit
def _tc_dense(u_p, i_p, oh_t, rtab, WT, bT):
    blk = 2048
    grid = (_BATCH // blk,)
    pad_spec = pl.BlockSpec((blk, _PAD), lambda i: (i, 0))
    emb_t_shape = jax.ShapeDtypeStruct((_ENC_HID, _BATCH), jnp.float32)
    return pl.pallas_call(
        _dense_body,
        grid=grid,
        in_specs=[
            pad_spec, pad_spec,
            pl.BlockSpec((_NRAT, blk), lambda i: (0, i)),
            pl.BlockSpec((_NRAT, _ENC_HID), lambda i: (0, 0)),
            pl.BlockSpec((_DEC_HID * _RNN_LAYERS, 3 * _ENC_HID), lambda i: (0, 0)),
            pl.BlockSpec((_DEC_HID * _RNN_LAYERS, 1), lambda i: (0, 0)),
        ],
        out_specs=[
            pl.BlockSpec((_DEC_HID * _RNN_LAYERS, blk), lambda i: (0, i)),
            pl.BlockSpec((_ENC_HID, blk), lambda i: (0, i)),
            pl.BlockSpec((_ENC_HID, blk), lambda i: (0, i)),
            pl.BlockSpec((_ENC_HID, blk), lambda i: (0, i)),
        ],
        out_shape=[
            jax.ShapeDtypeStruct((_DEC_HID * _RNN_LAYERS, _BATCH), jnp.float32),
            emb_t_shape, emb_t_shape, emb_t_shape,
        ],
    )(u_p, i_p, oh_t, rtab, WT, bT)


def kernel(user, item, rating, user_table, item_table, rating_table, W, b):
    user = user.astype(jnp.int32)
    item = item.astype(jnp.int32)
    rating = rating.astype(jnp.int32)
    u128, i128 = _tc_prep(user_table.T, item_table.T)
    u_p, i_p = _sc_gather(user, item, u128, i128)
    oh_t = (jnp.arange(_NRAT, dtype=jnp.int32)[:, None]
            == rating[None, :]).astype(jnp.float32)
    h_t, ue_t, ie_t, re_t = _tc_dense(u_p, i_p, oh_t, rating_table,
                                      W.T, b.reshape(-1, 1))
    hidden = h_t.T.reshape(-1, _DEC_HID, _RNN_LAYERS)
    return (hidden, ue_t.T, ie_t.T, re_t.T)


# speedup vs baseline: 1.1541x; 1.0243x over previous
import jax
import jax.numpy as jnp
from jax import lax
from jax.experimental import pallas as pl
from jax.experimental.pallas import tpu as pltpu
from jax.experimental.pallas import tpu_sc as plsc

_ENC_HID = 64
_DEC_HID = 64
_RNN_LAYERS = 2
_BATCH = 16384
_PAD = 128
_NRAT = 6

_NC = 2
_NS = 16
_NW = _NC * _NS
_BPW = _BATCH // _NW
_WIN = 256
_VG = 16


def _fire_window(tbl, idx, lo, buf, sem):
    copies = []
    for g in range(_WIN // _VG):
        iv = idx[pl.ds(lo + g * _VG, _VG)]
        copies.append(pltpu.async_copy(
            tbl.at[iv], buf.at[pl.ds(g * _VG, _VG)], sem))
    return copies


def _sc_gather_body(user_hbm, item_hbm, user_tbl, item_tbl,
                    u_out, i_out,
                    u_idx, i_idx, buf_a, buf_b, gsem_a, gsem_b, ssem):
    wid = lax.axis_index("s") * _NC + lax.axis_index("c")
    base = wid * _BPW

    pltpu.sync_copy(user_hbm.at[pl.ds(base, _BPW)], u_idx)
    pltpu.sync_copy(item_hbm.at[pl.ds(base, _BPW)], i_idx)

    ca = _fire_window(user_tbl, u_idx, 0, buf_a, gsem_a)
    cb = _fire_window(user_tbl, u_idx, _WIN, buf_b, gsem_b)
    for c in ca:
        c.wait()
    sa = pltpu.async_copy(buf_a, u_out.at[pl.ds(base, _WIN)], ssem)
    for c in cb:
        c.wait()
    sb = pltpu.async_copy(buf_b, u_out.at[pl.ds(base + _WIN, _WIN)], ssem)

    sa.wait()
    ca = _fire_window(item_tbl, i_idx, 0, buf_a, gsem_a)
    sb.wait()
    cb = _fire_window(item_tbl, i_idx, _WIN, buf_b, gsem_b)
    for c in ca:
        c.wait()
    sa = pltpu.async_copy(buf_a, i_out.at[pl.ds(base, _WIN)], ssem)
    for c in cb:
        c.wait()
    sb = pltpu.async_copy(buf_b, i_out.at[pl.ds(base + _WIN, _WIN)], ssem)
    sa.wait()
    sb.wait()


@jax.jit
def _sc_gather(user, item, user_t128, item_t128):
    padded = jax.ShapeDtypeStruct((_BATCH, _PAD), jnp.float32)
    mesh = plsc.VectorSubcoreMesh(core_axis_name="c", subcore_axis_name="s")
    return pl.kernel(
        _sc_gather_body,
        mesh=mesh,
        compiler_params=pltpu.CompilerParams(use_tc_tiling_on_sc=True),
        out_type=(padded, padded),
        scratch_types=[
            pltpu.VMEM((_BPW,), jnp.int32),
            pltpu.VMEM((_BPW,), jnp.int32),
            pltpu.VMEM((_WIN, _PAD), jnp.float32),
            pltpu.VMEM((_WIN, _PAD), jnp.float32),
            pltpu.SemaphoreType.DMA,
            pltpu.SemaphoreType.DMA,
            pltpu.SemaphoreType.DMA,
        ],
    )(user, item, user_t128, item_t128)


_PREP_LANES = 12800
_PREP_GRID = -(-100000 // _PREP_LANES)


def _prep_body(ut_ref, it_ref, u2_ref, i2_ref):
    u2_ref[:, :_ENC_HID] = ut_ref[...].T
    i2_ref[:, :_ENC_HID] = it_ref[...].T


@jax.jit
def _tc_prep(user_tT, item_tT):
    t2 = jax.ShapeDtypeStruct((100000, _PAD), jnp.float32)
    tin_spec = pl.BlockSpec((_ENC_HID, _PREP_LANES), lambda i: (0, i))
    tout_spec = pl.BlockSpec((_PREP_LANES, _PAD), lambda i: (i, 0))
    return pl.pallas_call(
        _prep_body,
        grid=(_PREP_GRID,),
        in_specs=[tin_spec, tin_spec],
        out_specs=[tout_spec, tout_spec],
        out_shape=[t2, t2],
    )(user_tT, item_tT)


def _dense_body(u_ref, i_ref, oh_ref, rtab_ref, wt_ref, b_ref,
                h_ref, ue_ref, ie_ref, re_ref):
    ue_t = u_ref[...][:, :_ENC_HID].T
    ie_t = i_ref[...][:, :_ENC_HID].T
    re_t = jnp.dot(rtab_ref[...].T, oh_ref[...],
                   preferred_element_type=jnp.float32)
    wt = wt_ref[...]
    acc = (jnp.dot(wt[:, :_ENC_HID], ue_t, preferred_element_type=jnp.float32)
           + jnp.dot(wt[:, _ENC_HID:2 * _ENC_HID], ie_t,
                     preferred_element_type=jnp.float32)
           + jnp.dot(wt[:, 2 * _ENC_HID:], re_t,
                     preferred_element_type=jnp.float32))
    h_ref[...] = jnp.tanh(acc + b_ref[...])
    ue_ref[...] = ue_t
    ie_ref[...] = ie_t
    re_ref[...] = re_t


@jax.jit
def _tc_dense(u_p, i_p, oh_t, rtab, WT, bT):
    blk = 2048
    grid = (_BATCH // blk,)
    pad_spec = pl.BlockSpec((blk, _PAD), lambda i: (i, 0))
    emb_t_shape = jax.ShapeDtypeStruct((_ENC_HID, _BATCH), jnp.float32)
    return pl.pallas_call(
        _dense_body,
        grid=grid,
        in_specs=[
            pad_spec, pad_spec,
            pl.BlockSpec((_NRAT, blk), lambda i: (0, i)),
            pl.BlockSpec((_NRAT, _ENC_HID), lambda i: (0, 0)),
            pl.BlockSpec((_DEC_HID * _RNN_LAYERS, 3 * _ENC_HID), lambda i: (0, 0)),
            pl.BlockSpec((_DEC_HID * _RNN_LAYERS, 1), lambda i: (0, 0)),
        ],
        out_specs=[
            pl.BlockSpec((_DEC_HID * _RNN_LAYERS, blk), lambda i: (0, i)),
            pl.BlockSpec((_ENC_HID, blk), lambda i: (0, i)),
            pl.BlockSpec((_ENC_HID, blk), lambda i: (0, i)),
            pl.BlockSpec((_ENC_HID, blk), lambda i: (0, i)),
        ],
        out_shape=[
            jax.ShapeDtypeStruct((_DEC_HID * _RNN_LAYERS, _BATCH), jnp.float32),
            emb_t_shape, emb_t_shape, emb_t_shape,
        ],
    )(u_p, i_p, oh_t, rtab, WT, bT)


def kernel(user, item, rating, user_table, item_table, rating_table, W, b):
    user = user.astype(jnp.int32)
    item = item.astype(jnp.int32)
    rating = rating.astype(jnp.int32)
    u128, i128 = _tc_prep(user_table.T, item_table.T)
    u_p, i_p = _sc_gather(user, item, u128, i128)
    oh_t = (jnp.arange(_NRAT, dtype=jnp.int32)[:, None]
            == rating[None, :]).astype(jnp.float32)
    h_t, ue_t, ie_t, re_t = _tc_dense(u_p, i_p, oh_t, rating_table,
                                      W.T, b.reshape(-1, 1))
    hidden = h_t.T.reshape(-1, _DEC_HID, _RNN_LAYERS)
    return (hidden, ue_t.T, ie_t.T, re_t.T)


# speedup vs baseline: 1.2888x; 1.1167x over previous
import jax
import jax.numpy as jnp
from jax import lax
from jax.experimental import pallas as pl
from jax.experimental.pallas import tpu as pltpu
from jax.experimental.pallas import tpu_sc as plsc

_ENC_HID = 64
_DEC_HID = 64
_RNN_LAYERS = 2
_BATCH = 16384
_PAD = 128
_NRAT = 6

_NC = 2
_NS = 16
_NW = _NC * _NS
_BPW = _BATCH // _NW
_WIN = 256
_VG = 16


def _fire_window(tbl, idx, lo, buf, sem):
    copies = []
    for g in range(_WIN // _VG):
        iv = idx[pl.ds(lo + g * _VG, _VG)]
        copies.append(pltpu.async_copy(
            tbl.at[iv], buf.at[pl.ds(g * _VG, _VG)], sem))
    return copies


def _sc_gather_body(user_hbm, item_hbm, user_tbl, item_tbl,
                    u_out, i_out,
                    u_idx, i_idx, buf_a, buf_b, gsem_a, gsem_b, ssem):
    wid = lax.axis_index("s") * _NC + lax.axis_index("c")
    base = wid * _BPW

    pltpu.sync_copy(user_hbm.at[pl.ds(base, _BPW)], u_idx)
    pltpu.sync_copy(item_hbm.at[pl.ds(base, _BPW)], i_idx)

    ca = _fire_window(user_tbl, u_idx, 0, buf_a, gsem_a)
    cb = _fire_window(user_tbl, u_idx, _WIN, buf_b, gsem_b)
    for c in ca:
        c.wait()
    sa = pltpu.async_copy(buf_a, u_out.at[pl.ds(base, _WIN)], ssem)
    for c in cb:
        c.wait()
    sb = pltpu.async_copy(buf_b, u_out.at[pl.ds(base + _WIN, _WIN)], ssem)

    sa.wait()
    ca = _fire_window(item_tbl, i_idx, 0, buf_a, gsem_a)
    sb.wait()
    cb = _fire_window(item_tbl, i_idx, _WIN, buf_b, gsem_b)
    for c in ca:
        c.wait()
    sa = pltpu.async_copy(buf_a, i_out.at[pl.ds(base, _WIN)], ssem)
    for c in cb:
        c.wait()
    sb = pltpu.async_copy(buf_b, i_out.at[pl.ds(base + _WIN, _WIN)], ssem)
    sa.wait()
    sb.wait()


@jax.jit
def _sc_gather(user, item, user_t128, item_t128):
    padded = jax.ShapeDtypeStruct((_BATCH, _PAD), jnp.float32)
    mesh = plsc.VectorSubcoreMesh(core_axis_name="c", subcore_axis_name="s")
    return pl.kernel(
        _sc_gather_body,
        mesh=mesh,
        compiler_params=pltpu.CompilerParams(use_tc_tiling_on_sc=True),
        out_type=(padded, padded),
        scratch_types=[
            pltpu.VMEM((_BPW,), jnp.int32),
            pltpu.VMEM((_BPW,), jnp.int32),
            pltpu.VMEM((_WIN, _PAD), jnp.float32),
            pltpu.VMEM((_WIN, _PAD), jnp.float32),
            pltpu.SemaphoreType.DMA,
            pltpu.SemaphoreType.DMA,
            pltpu.SemaphoreType.DMA,
        ],
    )(user, item, user_t128, item_t128)


_PREP_LANES = 12800
_PREP_GRID = -(-100000 // _PREP_LANES)


def _prep_body(ut_ref, it_ref, u2_ref, i2_ref):
    u2_ref[:, :_ENC_HID] = ut_ref[...].T
    i2_ref[:, :_ENC_HID] = it_ref[...].T


@jax.jit
def _tc_prep(user_tT, item_tT):
    t2 = jax.ShapeDtypeStruct((100000, _PAD), jnp.float32)
    tin_spec = pl.BlockSpec((_ENC_HID, _PREP_LANES), lambda i: (0, i))
    tout_spec = pl.BlockSpec((_PREP_LANES, _PAD), lambda i: (i, 0))
    return pl.pallas_call(
        _prep_body,
        grid=(_PREP_GRID,),
        in_specs=[tin_spec, tin_spec],
        out_specs=[tout_spec, tout_spec],
        out_shape=[t2, t2],
    )(user_tT, item_tT)


def _dense_body(u_ref, i_ref, oh_ref, rtab_ref, wt_ref, b_ref,
                h_ref, ue_ref, ie_ref, re_ref):
    ue_t = u_ref[...][:, :_ENC_HID].T
    ie_t = i_ref[...][:, :_ENC_HID].T
    re_t = jnp.dot(rtab_ref[...].T, oh_ref[...],
                   preferred_element_type=jnp.float32)
    wt = wt_ref[...]
    acc = (jnp.dot(wt[:, :_ENC_HID], ue_t, preferred_element_type=jnp.float32)
           + jnp.dot(wt[:, _ENC_HID:2 * _ENC_HID], ie_t,
                     preferred_element_type=jnp.float32)
           + jnp.dot(wt[:, 2 * _ENC_HID:], re_t,
                     preferred_element_type=jnp.float32))
    h_ref[...] = jnp.tanh(acc + b_ref[...]).reshape(
        _DEC_HID, _RNN_LAYERS, acc.shape[1])
    ue_ref[...] = ue_t
    ie_ref[...] = ie_t
    re_ref[...] = re_t


@jax.jit
def _tc_dense(u_p, i_p, oh_t, rtab, WT, bT):
    blk = 2048
    grid = (_BATCH // blk,)
    pad_spec = pl.BlockSpec((blk, _PAD), lambda i: (i, 0))
    emb_t_shape = jax.ShapeDtypeStruct((_ENC_HID, _BATCH), jnp.float32)
    return pl.pallas_call(
        _dense_body,
        grid=grid,
        in_specs=[
            pad_spec, pad_spec,
            pl.BlockSpec((_NRAT, blk), lambda i: (0, i)),
            pl.BlockSpec((_NRAT, _ENC_HID), lambda i: (0, 0)),
            pl.BlockSpec((_DEC_HID * _RNN_LAYERS, 3 * _ENC_HID), lambda i: (0, 0)),
            pl.BlockSpec((_DEC_HID * _RNN_LAYERS, 1), lambda i: (0, 0)),
        ],
        out_specs=[
            pl.BlockSpec((_DEC_HID, _RNN_LAYERS, blk), lambda i: (0, 0, i)),
            pl.BlockSpec((_ENC_HID, blk), lambda i: (0, i)),
            pl.BlockSpec((_ENC_HID, blk), lambda i: (0, i)),
            pl.BlockSpec((_ENC_HID, blk), lambda i: (0, i)),
        ],
        out_shape=[
            jax.ShapeDtypeStruct((_DEC_HID, _RNN_LAYERS, _BATCH), jnp.float32),
            emb_t_shape, emb_t_shape, emb_t_shape,
        ],
    )(u_p, i_p, oh_t, rtab, WT, bT)


def kernel(user, item, rating, user_table, item_table, rating_table, W, b):
    user = user.astype(jnp.int32)
    item = item.astype(jnp.int32)
    rating = rating.astype(jnp.int32)
    u128, i128 = _tc_prep(user_table.T, item_table.T)
    u_p, i_p = _sc_gather(user, item, u128, i128)
    oh_t = (jnp.arange(_NRAT, dtype=jnp.int32)[:, None]
            == rating[None, :]).astype(jnp.float32)
    h_t, ue_t, ie_t, re_t = _tc_dense(u_p, i_p, oh_t, rating_table,
                                      W.T, b.reshape(-1, 1))
    hidden = h_t.transpose(2, 0, 1)
    return (hidden, ue_t.T, ie_t.T, re_t.T)
